# Initial kernel scaffold; baseline (speedup 1.0000x reference)
import jax
import jax.numpy as jnp
from jax.experimental import pallas as pl


def kernel(z, Wr, br, W1, b1, W2, b2, Wproj, ln_g, ln_b, Wo, bo):
    raise NotImplementedError("write your pallas kernel here")


# speedup vs baseline: 2.7738x; 2.7738x over previous
import functools
import math

import jax
import jax.numpy as jnp
from jax.experimental import pallas as pl
from jax.experimental.pallas import tpu as pltpu

_E = 8
_K = 2
_INV_SQRT2 = 1.0 / math.sqrt(2.0)


def _moe_block(z_ref, wrT_ref, br_ref, waT_ref, b1_ref, w2T_ref, b2_ref,
               g_ref, be_ref, wo_ref, bo_ref,
               y_ref, probs_ref, idx_ref, imp_ref, load_ref):
    B = z_ref.shape[0]
    H = b1_ref.shape[1]
    zb = z_ref[...]

    logits = jnp.dot(zb, wrT_ref[...], preferred_element_type=jnp.float32)
    logits = logits + br_ref[...]
    m = jnp.max(logits, axis=-1, keepdims=True)
    ex = jnp.exp(logits - m)
    probs = ex / jnp.sum(ex, axis=-1, keepdims=True)

    i1 = jnp.argmax(probs, axis=-1)
    p1 = jnp.max(probs, axis=-1)
    eids = jax.lax.broadcasted_iota(jnp.int32, (B, _E), 1)
    probs_m = jnp.where(eids == i1[:, None], -1.0, probs)
    i2 = jnp.argmax(probs_m, axis=-1)
    p2 = jnp.max(probs_m, axis=-1)
    denom = jnp.maximum(p1 + p2, 1e-8)
    w1 = (p1 / denom)[:, None]
    w2 = (p2 / denom)[:, None]

    probs_ref[...] = probs
    idx_ref[...] = jnp.stack([i1, i2], axis=1).astype(jnp.int32)

    acc = jnp.zeros((B, 1), dtype=jnp.float32)
    for e in range(_E):
        t = jnp.dot(zb, waT_ref[e], preferred_element_type=jnp.float32)
        hpre = t[:, :H] + b1_ref[e][None, :]
        h = 0.5 * hpre * (1.0 + jax.lax.erf(hpre * _INV_SQRT2))
        h2 = jnp.dot(h, w2T_ref[e], preferred_element_type=jnp.float32)
        r = h2 + b2_ref[e][None, :] + t[:, H:]
        mu = jnp.mean(r, axis=-1, keepdims=True)
        c = r - mu
        var = jnp.mean(c * c, axis=-1, keepdims=True)
        rn = c * jax.lax.rsqrt(var + 1e-5) * g_ref[e][None, :] + be_ref[e][None, :]
        ye = jnp.sum(rn * wo_ref[e][None, :], axis=-1, keepdims=True) + bo_ref[0, e]
        wsel = jnp.where(i1 == e, w1[:, 0], 0.0) + jnp.where(i2 == e, w2[:, 0], 0.0)
        acc = acc + ye * wsel[:, None]
    y_ref[...] = acc

    @pl.when(pl.program_id(0) == 0)
    def _init():
        imp_ref[...] = jnp.zeros_like(imp_ref)
        load_ref[...] = jnp.zeros_like(load_ref)

    imp_ref[...] += jnp.sum(probs, axis=0, keepdims=True)
    hit = (eids == i1[:, None]) | (eids == i2[:, None])
    load_ref[...] += jnp.sum(hit.astype(jnp.float32), axis=0, keepdims=True)


def kernel(z, Wr, br, W1, b1, W2, b2, Wproj, ln_g, ln_b, Wo, bo):
    N, D = z.shape
    E, H = b1.shape
    B = 1024
    nblk = N // B

    waT = jnp.concatenate([W1, Wproj], axis=1).transpose(0, 2, 1)
    w2T = W2.transpose(0, 2, 1)
    wrT = Wr.T
    wo = Wo[:, 0, :]

    full = lambda *shape: pl.BlockSpec(shape, lambda i: (0,) * len(shape))
    out_shapes = (
        jax.ShapeDtypeStruct((N, 1), jnp.float32),
        jax.ShapeDtypeStruct((N, E), jnp.float32),
        jax.ShapeDtypeStruct((N, _K), jnp.int32),
        jax.ShapeDtypeStruct((1, E), jnp.float32),
        jax.ShapeDtypeStruct((1, E), jnp.float32),
    )
    y, probs, idx, imp, load = pl.pallas_call(
        _moe_block,
        grid=(nblk,),
        in_specs=[
            pl.BlockSpec((B, D), lambda i: (i, 0)),
            full(D, E),
            full(1, E),
            full(E, D, 2 * H),
            full(E, H),
            full(E, H, H),
            full(E, H),
            full(E, H),
            full(E, H),
            full(E, H),
            full(1, E),
        ],
        out_specs=(
            pl.BlockSpec((B, 1), lambda i: (i, 0)),
            pl.BlockSpec((B, E), lambda i: (i, 0)),
            pl.BlockSpec((B, _K), lambda i: (i, 0)),
            pl.BlockSpec((1, E), lambda i: (0, 0)),
            pl.BlockSpec((1, E), lambda i: (0, 0)),
        ),
        out_shape=out_shapes,
        compiler_params=pltpu.CompilerParams(
            dimension_semantics=("arbitrary",),
        ),
    )(z, wrT, br.reshape(1, E), waT, b1, w2T, b2, ln_g, ln_b, wo,
      bo.reshape(1, E))

    inv_n = 1.0 / N
    return (y, probs, idx, (imp[0] * inv_n), (load[0] * inv_n))


# speedup vs baseline: 3.0124x; 1.0860x over previous
import functools
import math

import jax
import jax.numpy as jnp
from jax.experimental import pallas as pl
from jax.experimental.pallas import tpu as pltpu

_E = 8
_K = 2
_INV_SQRT2 = 1.0 / math.sqrt(2.0)


def _moe_block(z_ref, wrT_ref, br_ref, waT_ref, b1_ref, w2T_ref, b2_ref,
               g_ref, be_ref, wo_ref, bo_ref,
               y_ref, probs_ref, idx_ref, imp_ref, load_ref):
    B = z_ref.shape[0]
    H = b1_ref.shape[1]
    zb = z_ref[...]

    logits = jnp.dot(zb, wrT_ref[...], preferred_element_type=jnp.float32)
    logits = logits + br_ref[...]
    m = jnp.max(logits, axis=-1, keepdims=True)
    ex = jnp.exp(logits - m)
    probs = ex / jnp.sum(ex, axis=-1, keepdims=True)

    i1 = jnp.argmax(probs, axis=-1)
    p1 = jnp.max(probs, axis=-1)
    eids = jax.lax.broadcasted_iota(jnp.int32, (B, _E), 1)
    probs_m = jnp.where(eids == i1[:, None], -1.0, probs)
    i2 = jnp.argmax(probs_m, axis=-1)
    p2 = jnp.max(probs_m, axis=-1)
    denom = jnp.maximum(p1 + p2, 1e-8)
    w1 = (p1 / denom)[:, None]
    w2 = (p2 / denom)[:, None]

    probs_ref[...] = probs
    idx_ref[...] = jnp.stack([i1, i2], axis=1).astype(jnp.int32)

    zb16 = zb.astype(jnp.bfloat16)
    acc = jnp.zeros((B, 1), dtype=jnp.float32)
    for e in range(_E):
        t = jnp.dot(zb16, waT_ref[e], preferred_element_type=jnp.float32)
        hpre = t[:, :H] + b1_ref[e][None, :]
        h = 0.5 * hpre * (1.0 + jax.lax.erf(hpre * _INV_SQRT2))
        h2 = jnp.dot(h.astype(jnp.bfloat16), w2T_ref[e],
                     preferred_element_type=jnp.float32)
        r = h2 + b2_ref[e][None, :] + t[:, H:]
        mu = jnp.mean(r, axis=-1, keepdims=True)
        c = r - mu
        var = jnp.mean(c * c, axis=-1, keepdims=True)
        rn = c * jax.lax.rsqrt(var + 1e-5) * g_ref[e][None, :] + be_ref[e][None, :]
        ye = jnp.sum(rn * wo_ref[e][None, :], axis=-1, keepdims=True) + bo_ref[0, e]
        wsel = jnp.where(i1 == e, w1[:, 0], 0.0) + jnp.where(i2 == e, w2[:, 0], 0.0)
        acc = acc + ye * wsel[:, None]
    y_ref[...] = acc

    @pl.when(pl.program_id(0) == 0)
    def _init():
        imp_ref[...] = jnp.zeros_like(imp_ref)
        load_ref[...] = jnp.zeros_like(load_ref)

    imp_ref[...] += jnp.sum(probs, axis=0, keepdims=True)
    hit = (eids == i1[:, None]) | (eids == i2[:, None])
    load_ref[...] += jnp.sum(hit.astype(jnp.float32), axis=0, keepdims=True)


def kernel(z, Wr, br, W1, b1, W2, b2, Wproj, ln_g, ln_b, Wo, bo):
    N, D = z.shape
    E, H = b1.shape
    B = 1024
    nblk = N // B

    waT = jnp.concatenate([W1, Wproj], axis=1).transpose(0, 2, 1)
    waT = waT.astype(jnp.bfloat16)
    w2T = W2.transpose(0, 2, 1).astype(jnp.bfloat16)
    wrT = Wr.T
    wo = Wo[:, 0, :]

    full = lambda *shape: pl.BlockSpec(shape, lambda i: (0,) * len(shape))
    out_shapes = (
        jax.ShapeDtypeStruct((N, 1), jnp.float32),
        jax.ShapeDtypeStruct((N, E), jnp.float32),
        jax.ShapeDtypeStruct((N, _K), jnp.int32),
        jax.ShapeDtypeStruct((1, E), jnp.float32),
        jax.ShapeDtypeStruct((1, E), jnp.float32),
    )
    y, probs, idx, imp, load = pl.pallas_call(
        _moe_block,
        grid=(nblk,),
        in_specs=[
            pl.BlockSpec((B, D), lambda i: (i, 0)),
            full(D, E),
            full(1, E),
            full(E, D, 2 * H),
            full(E, H),
            full(E, H, H),
            full(E, H),
            full(E, H),
            full(E, H),
            full(E, H),
            full(1, E),
        ],
        out_specs=(
            pl.BlockSpec((B, 1), lambda i: (i, 0)),
            pl.BlockSpec((B, E), lambda i: (i, 0)),
            pl.BlockSpec((B, _K), lambda i: (i, 0)),
            pl.BlockSpec((1, E), lambda i: (0, 0)),
            pl.BlockSpec((1, E), lambda i: (0, 0)),
        ),
        out_shape=out_shapes,
        compiler_params=pltpu.CompilerParams(
            dimension_semantics=("arbitrary",),
        ),
    )(z, wrT, br.reshape(1, E), waT, b1, w2T, b2, ln_g, ln_b, wo,
      bo.reshape(1, E))

    inv_n = 1.0 / N
    return (y, probs, idx, (imp[0] * inv_n), (load[0] * inv_n))


# speedup vs baseline: 4.4764x; 1.4860x over previous
import functools
import math

import jax
import jax.numpy as jnp
from jax.experimental import pallas as pl
from jax.experimental.pallas import tpu as pltpu

_E = 8
_K = 2
_INV_SQRT2 = 1.0 / math.sqrt(2.0)


def _moe_block(z_ref, wrT_ref, br_ref, w1T_ref, wpT_ref, b1_ref, w2T_ref,
               b2_ref, u_ref, su_ref, s0_ref, y_ref, probs_ref, idx_ref,
               imp_ref, load_ref):
    B = z_ref.shape[0]
    H = b1_ref.shape[1]
    zb = z_ref[...]

    logits = jnp.dot(zb, wrT_ref[...], preferred_element_type=jnp.float32)
    logits = logits + br_ref[...]
    m = jnp.max(logits, axis=-1, keepdims=True)
    ex = jnp.exp(logits - m)
    probs = ex / jnp.sum(ex, axis=-1, keepdims=True)

    i1 = jnp.argmax(probs, axis=-1)
    p1 = jnp.max(probs, axis=-1)
    eids = jax.lax.broadcasted_iota(jnp.int32, (B, _E), 1)
    probs_m = jnp.where(eids == i1[:, None], -1.0, probs)
    i2 = jnp.argmax(probs_m, axis=-1)
    p2 = jnp.max(probs_m, axis=-1)
    denom = jnp.maximum(p1 + p2, 1e-8)
    w1 = (p1 / denom)[:, None]
    w2 = (p2 / denom)[:, None]

    probs_ref[...] = probs
    idx_ref[...] = jnp.stack([i1, i2], axis=1).astype(jnp.int32)

    zb16 = zb.astype(jnp.bfloat16)
    inv_h = 1.0 / H
    acc = jnp.zeros((B,), dtype=jnp.float32)
    for e in range(_E):
        hpre = jnp.dot(zb16, w1T_ref[e], preferred_element_type=jnp.float32)
        hpre = hpre + b1_ref[e][None, :]
        xp = jnp.dot(zb16, wpT_ref[e], preferred_element_type=jnp.float32)
        h = 0.5 * hpre * (1.0 + jax.lax.erf(hpre * _INV_SQRT2))
        h2 = jnp.dot(h.astype(jnp.bfloat16), w2T_ref[e],
                     preferred_element_type=jnp.float32)
        r = h2 + b2_ref[e][None, :] + xp
        s1 = jnp.sum(r, axis=-1)
        s2 = jnp.sum(r * r, axis=-1)
        sru = jnp.sum(r * u_ref[e][None, :], axis=-1)
        mu = s1 * inv_h
        var = s2 * inv_h - mu * mu
        inv = jax.lax.rsqrt(var + 1e-5)
        ye = inv * (sru - mu * su_ref[0, e]) + s0_ref[0, e]
        wsel = jnp.where(i1 == e, w1[:, 0], 0.0) + jnp.where(i2 == e, w2[:, 0], 0.0)
        acc = acc + ye * wsel
    y_ref[...] = acc[:, None]

    @pl.when(pl.program_id(0) == 0)
    def _init():
        imp_ref[...] = jnp.zeros_like(imp_ref)
        load_ref[...] = jnp.zeros_like(load_ref)

    imp_ref[...] += jnp.sum(probs, axis=0, keepdims=True)
    hit = (eids == i1[:, None]) | (eids == i2[:, None])
    load_ref[...] += jnp.sum(hit.astype(jnp.float32), axis=0, keepdims=True)


def kernel(z, Wr, br, W1, b1, W2, b2, Wproj, ln_g, ln_b, Wo, bo):
    N, D = z.shape
    E, H = b1.shape
    B = 1024
    nblk = N // B

    w1T = W1.transpose(0, 2, 1).astype(jnp.bfloat16)
    wpT = Wproj.transpose(0, 2, 1).astype(jnp.bfloat16)
    w2T = W2.transpose(0, 2, 1).astype(jnp.bfloat16)
    wrT = Wr.T
    wo = Wo[:, 0, :]
    u = ln_g * wo
    su = jnp.sum(u, axis=1).reshape(1, E)
    s0 = (jnp.sum(ln_b * wo, axis=1) + bo[:, 0]).reshape(1, E)

    full = lambda *shape: pl.BlockSpec(shape, lambda i: (0,) * len(shape))
    out_shapes = (
        jax.ShapeDtypeStruct((N, 1), jnp.float32),
        jax.ShapeDtypeStruct((N, E), jnp.float32),
        jax.ShapeDtypeStruct((N, _K), jnp.int32),
        jax.ShapeDtypeStruct((1, E), jnp.float32),
        jax.ShapeDtypeStruct((1, E), jnp.float32),
    )
    y, probs, idx, imp, load = pl.pallas_call(
        _moe_block,
        grid=(nblk,),
        in_specs=[
            pl.BlockSpec((B, D), lambda i: (i, 0)),
            full(D, E),
            full(1, E),
            full(E, D, H),
            full(E, D, H),
            full(E, H),
            full(E, H, H),
            full(E, H),
            full(E, H),
            full(1, E),
            full(1, E),
        ],
        out_specs=(
            pl.BlockSpec((B, 1), lambda i: (i, 0)),
            pl.BlockSpec((B, E), lambda i: (i, 0)),
            pl.BlockSpec((B, _K), lambda i: (i, 0)),
            pl.BlockSpec((1, E), lambda i: (0, 0)),
            pl.BlockSpec((1, E), lambda i: (0, 0)),
        ),
        out_shape=out_shapes,
        compiler_params=pltpu.CompilerParams(
            dimension_semantics=("arbitrary",),
        ),
    )(z, wrT, br.reshape(1, E), w1T, wpT, b1, w2T, b2, u, su, s0)

    inv_n = 1.0 / N
    return (y, probs, idx, (imp[0] * inv_n), (load[0] * inv_n))
